# baseline (speedup 1.0000x reference)
import functools

import jax
import jax.numpy as jnp
from jax import lax
from jax.experimental import pallas as pl
from jax.experimental.pallas import tpu as pltpu
from jax.experimental.pallas import tpu_sc as plsc

VOCAB = 100000
DIM = 64
BATCH = 4
SEQ = 8192
TOTAL = BATCH * SEQ
NW = 32
PER_W = TOTAL // NW
CH = 256
NST = PER_W // CH
LANES = 16
VPR = DIM // LANES

_mesh = plsc.VectorSubcoreMesh(core_axis_name="c", subcore_axis_name="s")


@functools.partial(
    pl.kernel,
    mesh=_mesh,
    out_type=jax.ShapeDtypeStruct((BATCH, SEQ, DIM), jnp.float32),
    scratch_types=[
        pltpu.VMEM((PER_W,), jnp.int32),
        pltpu.VMEM((2, CH, DIM), jnp.float32),
        pltpu.VMEM((CH, DIM), jnp.float32),
        pltpu.SemaphoreType.DMA,
        pltpu.SemaphoreType.DMA,
        pltpu.SemaphoreType.DMA,
        pltpu.SemaphoreType.DMA,
    ],
)
def _embed(ids_hbm, tok_hbm, pos_hbm, out_hbm, idx_v, rows2, pos_v,
           gsem0, gsem1, psem, osem):
    wid = lax.axis_index("s") * 2 + lax.axis_index("c")
    b = wid // (SEQ // PER_W)
    s0 = (wid % (SEQ // PER_W)) * PER_W

    pltpu.sync_copy(ids_hbm.at[b, pl.ds(s0, PER_W)], idx_v)

    gsems = (gsem0, gsem1)
    lane_iota = lax.iota(jnp.int32, LANES)

    def fire_gathers(q, buf, sem):
        def fire_group(g, _):
            v = idx_v[pl.ds(q * CH + g * LANES, LANES)]
            for l in range(LANES):
                pltpu.async_copy(tok_hbm.at[v[l]], buf.at[g * LANES + l], sem)
            return ()
        lax.fori_loop(0, CH // LANES, fire_group, (), unroll=2)

    def drain_gathers(buf, sem):
        def drain(r, _):
            pltpu.make_async_copy(tok_hbm.at[0], buf.at[0], sem).wait()
            return ()
        lax.fori_loop(0, CH, drain, (), unroll=8)

    def fire_pos(q):
        pltpu.async_copy(pos_hbm.at[pl.ds(s0 + q * CH, CH)], pos_v, psem)

    def wait_pos():
        pltpu.make_async_copy(pos_hbm.at[pl.ds(s0, CH)], pos_v, psem).wait()

    def fire_out(q, buf):
        pltpu.async_copy(buf, out_hbm.at[b, pl.ds(s0 + q * CH, CH)], osem)

    def wait_out(q, buf):
        pltpu.make_async_copy(
            buf, out_hbm.at[b, pl.ds(s0 + q * CH, CH)], osem).wait()

    fire_gathers(0, rows2.at[0], gsems[0])
    fire_pos(0)

    for q in range(NST):
        cur = q % 2
        if q >= 2:
            wait_out(q - 2, rows2.at[cur])
        if q + 1 < NST:
            fire_gathers(q + 1, rows2.at[(q + 1) % 2], gsems[(q + 1) % 2])
        drain_gathers(rows2.at[cur], gsems[cur])
        wait_pos()

        buf = rows2.at[cur]

        def add_row(r, _):
            for j in range(VPR):
                sl = pl.ds(j * LANES, LANES)
                buf[r, sl] = buf[r, sl] + pos_v[r, sl]
            return ()

        lax.fori_loop(0, CH, add_row, (), unroll=4)

        if q + 1 < NST:
            fire_pos(q + 1)
        fire_out(q, buf)

    wait_out(NST - 2, rows2.at[(NST - 2) % 2])
    wait_out(NST - 1, rows2.at[(NST - 1) % 2])


def kernel(input_ids, token_table, pos_table):
    return _embed(input_ids, token_table, pos_table)


# speedup vs baseline: 1.0164x; 1.0164x over previous
import functools

import jax
import jax.numpy as jnp
from jax import lax
from jax.experimental import pallas as pl
from jax.experimental.pallas import tpu as pltpu
from jax.experimental.pallas import tpu_sc as plsc

VOCAB = 100000
DIM = 64
BATCH = 4
SEQ = 8192
TOTAL = BATCH * SEQ
NW = 32
PER_W = TOTAL // NW
CH = 256
NST = PER_W // CH
LANES = 16
VPR = DIM // LANES

_mesh = plsc.VectorSubcoreMesh(core_axis_name="c", subcore_axis_name="s")


@functools.partial(
    pl.kernel,
    mesh=_mesh,
    out_type=jax.ShapeDtypeStruct((BATCH, SEQ, DIM), jnp.float32),
    scratch_types=[
        pltpu.VMEM((PER_W,), jnp.int32),
        pltpu.VMEM((2, CH, DIM), jnp.float32),
        pltpu.VMEM((CH, DIM), jnp.float32),
        pltpu.SemaphoreType.DMA,
        pltpu.SemaphoreType.DMA,
        pltpu.SemaphoreType.DMA,
        pltpu.SemaphoreType.DMA,
    ],
)
def _embed(ids_hbm, tok_hbm, pos_hbm, out_hbm, idx_v, rows2, pos_v,
           gsem0, gsem1, psem, osem):
    wid = lax.axis_index("s") * 2 + lax.axis_index("c")
    b = wid // (SEQ // PER_W)
    s0 = (wid % (SEQ // PER_W)) * PER_W

    pltpu.sync_copy(ids_hbm.at[b, pl.ds(s0, PER_W)], idx_v)

    gsems = (gsem0, gsem1)
    lane_iota = lax.iota(jnp.int32, LANES)

    def fire_gathers(q, buf, sem):
        def fire_group(g, _):
            v = idx_v[pl.ds(q * CH + g * LANES, LANES)]
            for l in range(LANES):
                pltpu.async_copy(tok_hbm.at[v[l]], buf.at[g * LANES + l], sem)
            return ()
        lax.fori_loop(0, CH // LANES, fire_group, ())

    def drain_gathers(buf, sem):
        def drain(r, _):
            pltpu.make_async_copy(tok_hbm.at[0], buf.at[0], sem).wait()
            return ()
        lax.fori_loop(0, CH, drain, (), unroll=4)

    def fire_pos(q):
        pltpu.async_copy(pos_hbm.at[pl.ds(s0 + q * CH, CH)], pos_v, psem)

    def wait_pos():
        pltpu.make_async_copy(pos_hbm.at[pl.ds(s0, CH)], pos_v, psem).wait()

    def fire_out(q, buf):
        pltpu.async_copy(buf, out_hbm.at[b, pl.ds(s0 + q * CH, CH)], osem)

    def wait_out(q, buf):
        pltpu.make_async_copy(
            buf, out_hbm.at[b, pl.ds(s0 + q * CH, CH)], osem).wait()

    fire_gathers(0, rows2.at[0], gsems[0])
    fire_pos(0)

    for q in range(NST):
        cur = q % 2
        if q >= 2:
            wait_out(q - 2, rows2.at[cur])
        if q + 1 < NST:
            fire_gathers(q + 1, rows2.at[(q + 1) % 2], gsems[(q + 1) % 2])
        drain_gathers(rows2.at[cur], gsems[cur])
        wait_pos()

        buf = rows2.at[cur]

        def add_row(r, _):
            for j in range(VPR):
                sl = pl.ds(j * LANES, LANES)
                buf[r, sl] = buf[r, sl] + pos_v[r, sl]
            return ()

        lax.fori_loop(0, CH, add_row, (), unroll=2)

        if q + 1 < NST:
            fire_pos(q + 1)
        fire_out(q, buf)

    wait_out(NST - 2, rows2.at[(NST - 2) % 2])
    wait_out(NST - 1, rows2.at[(NST - 1) % 2])


def kernel(input_ids, token_table, pos_table):
    return _embed(input_ids, token_table, pos_table)
